# baseline (speedup 1.0000x reference)
import jax
import jax.numpy as jnp
from jax.experimental import pallas as pl

_BB = 8


def _caps_body(x_ref, dist_ref, feat_ref):
    xb = x_ref[...]
    sumsq = jnp.sum(xb * xb, axis=1)
    norm = jnp.sqrt(sumsq)
    mx = jnp.max(norm, axis=1, keepdims=True)
    e = jnp.exp(norm - mx)
    dist_ref[...] = e / jnp.sum(e, axis=1, keepdims=True)
    d_iota = jax.lax.broadcasted_iota(jnp.int32, norm.shape, 1)
    idx = jnp.min(jnp.where(norm == mx, d_iota, norm.shape[1]), axis=1,
                  keepdims=True)
    mask = d_iota == idx
    feat_ref[...] = jnp.where(mask[:, None, :], xb, 0.0)


def kernel(x):
    B, C, D = x.shape
    dist, feat = pl.pallas_call(
        _caps_body,
        grid=(B // _BB,),
        in_specs=[pl.BlockSpec((_BB, C, D), lambda i: (i, 0, 0))],
        out_specs=[
            pl.BlockSpec((_BB, D), lambda i: (i, 0)),
            pl.BlockSpec((_BB, C, D), lambda i: (i, 0, 0)),
        ],
        out_shape=[
            jax.ShapeDtypeStruct((B, D), x.dtype),
            jax.ShapeDtypeStruct((B, C, D), x.dtype),
        ],
    )(x)
    return dist, feat.reshape(B, C * D)


# speedup vs baseline: 2.8682x; 2.8682x over previous
import jax
import jax.numpy as jnp
from jax.experimental import pallas as pl

_BB = 64


def _caps_body(x_ref, dist_ref, feat_ref):
    xb = x_ref[...]
    BB, C, D = xb.shape
    sumsq = jnp.sum(xb * xb, axis=1)
    norm = jnp.sqrt(sumsq)
    mx = jnp.max(norm, axis=1, keepdims=True)
    e = jnp.exp(norm - mx)
    dist_ref[...] = e / jnp.sum(e, axis=1, keepdims=True)
    d_iota = jax.lax.broadcasted_iota(jnp.int32, norm.shape, 1)
    idx = jnp.min(jnp.where(norm == mx, d_iota, D), axis=1,
                  keepdims=True)
    mask = d_iota == idx
    masked = jnp.where(mask[:, None, :], xb, 0.0)
    feat_ref[...] = masked.reshape(BB, C * D)


def kernel(x):
    B, C, D = x.shape
    dist, feat = pl.pallas_call(
        _caps_body,
        grid=(B // _BB,),
        in_specs=[pl.BlockSpec((_BB, C, D), lambda i: (i, 0, 0))],
        out_specs=[
            pl.BlockSpec((_BB, D), lambda i: (i, 0)),
            pl.BlockSpec((_BB, C * D), lambda i: (i, 0)),
        ],
        out_shape=[
            jax.ShapeDtypeStruct((B, D), x.dtype),
            jax.ShapeDtypeStruct((B, C * D), x.dtype),
        ],
    )(x)
    return dist, feat
